# Initial kernel scaffold; baseline (speedup 1.0000x reference)
import jax
import jax.numpy as jnp
from jax.experimental import pallas as pl


def kernel(h_src, h_dst, node_type_src_argmax, node_type_dst_argmax, edge_type_argmax, edge_type_w, edge_type_b, event_maps, inference):
    raise NotImplementedError("write your pallas kernel here")


# speedup vs baseline: 7.2154x; 7.2154x over previous
import functools

import jax
import jax.numpy as jnp
from jax.experimental import pallas as pl
from jax.experimental.pallas import tpu as pltpu

_NUM_NODE_TYPES = 4
_NUM_PAIRS = 16
_NUM_GLOBAL = 32
_NUM_LOCAL = 8
_PAD = 128


def _fused_kernel(hs_ref, hd_ref, src_ref, dst_ref, etype_ref, w_ref, b_ref,
                  em_ref, out_ref, acc_ref, *, nblk):
    i = pl.program_id(0)

    @pl.when(i == 0)
    def _init():
        acc_ref[...] = jnp.zeros_like(acc_ref)

    h = hs_ref[...] * hd_ref[...]
    logits = jax.lax.dot_general(
        h, w_ref[...], (((1,), (0,)), ((), ())),
        preferred_element_type=jnp.float32,
        precision=jax.lax.Precision.HIGHEST) + b_ref[...]

    bsz = logits.shape[0]
    pair = src_ref[...] * _NUM_NODE_TYPES + dst_ref[...]
    col = jax.lax.broadcasted_iota(jnp.int32, (bsz, _PAD), 1)

    g = pair * _NUM_GLOBAL + etype_ref[...]
    tcol = jax.lax.broadcasted_iota(jnp.int32, (bsz, _NUM_PAIRS * _NUM_GLOBAL), 1)
    table = em_ref[0:1, :]
    label = jnp.sum(jnp.where(tcol == g, table, 0), axis=1, keepdims=True)

    base = pair * _NUM_LOCAL
    in_pair = (col >= base) & (col < base + _NUM_LOCAL)
    ml = jnp.where(in_pair, logits, -1e30)
    m = jnp.max(ml, axis=1, keepdims=True)
    lse = m + jnp.log(jnp.sum(jnp.exp(ml - m), axis=1, keepdims=True))
    picked = jnp.sum(jnp.where(col == base + label, logits, 0.0),
                     axis=1, keepdims=True)
    per_ex = lse - picked

    pbin = col == pair
    sums = jnp.sum(jnp.where(pbin, per_ex, 0.0), axis=0, keepdims=True)
    cnts = jnp.sum(pbin.astype(jnp.float32), axis=0, keepdims=True)
    acc_ref[...] += jnp.concatenate([sums, cnts], axis=0)

    @pl.when(i == nblk - 1)
    def _finish():
        tot = acc_ref[0:1, :]
        cnt = acc_ref[1:2, :]
        means = tot / jnp.maximum(cnt, 1.0)
        w = (cnt > 0.0).astype(jnp.float32)
        loss = jnp.sum(means * w) / jnp.maximum(jnp.sum(w), 1.0)
        out_ref[...] = jnp.reshape(loss, (1, 1))


@functools.partial(jax.jit, static_argnames=())
def _run(h_src, h_dst, src_i, dst_i, etype_i, w_all, b_all, em_flat):
    n, d = h_src.shape
    bsz = 1024
    nblk = n // bsz
    out = pl.pallas_call(
        functools.partial(_fused_kernel, nblk=nblk),
        grid=(nblk,),
        in_specs=[
            pl.BlockSpec((bsz, d), lambda i: (i, 0)),
            pl.BlockSpec((bsz, d), lambda i: (i, 0)),
            pl.BlockSpec((bsz, 1), lambda i: (i, 0)),
            pl.BlockSpec((bsz, 1), lambda i: (i, 0)),
            pl.BlockSpec((bsz, 1), lambda i: (i, 0)),
            pl.BlockSpec((d, _PAD), lambda i: (0, 0)),
            pl.BlockSpec((1, _PAD), lambda i: (0, 0)),
            pl.BlockSpec((8, _NUM_PAIRS * _NUM_GLOBAL), lambda i: (0, 0)),
        ],
        out_specs=pl.BlockSpec((1, 1), lambda i: (0, 0)),
        out_shape=jax.ShapeDtypeStruct((1, 1), jnp.float32),
        scratch_shapes=[pltpu.VMEM((2, _PAD), jnp.float32)],
        compiler_params=pltpu.CompilerParams(
            dimension_semantics=("arbitrary",)),
    )(h_src, h_dst, src_i, dst_i, etype_i, w_all, b_all, em_flat)
    return out[0, 0]


def kernel(h_src, h_dst, node_type_src_argmax, node_type_dst_argmax,
           edge_type_argmax, edge_type_w, edge_type_b, event_maps, inference):
    n = h_src.shape[0]
    src_i = node_type_src_argmax.astype(jnp.int32).reshape(n, 1)
    dst_i = node_type_dst_argmax.astype(jnp.int32).reshape(n, 1)
    etype_i = edge_type_argmax.astype(jnp.int32).reshape(n, 1)
    w_all = jnp.transpose(edge_type_w, (1, 0, 2)).reshape(h_src.shape[1], _PAD)
    b_all = edge_type_b.reshape(1, _PAD)
    em_flat = jnp.broadcast_to(
        event_maps.astype(jnp.int32).reshape(1, _NUM_PAIRS * _NUM_GLOBAL),
        (8, _NUM_PAIRS * _NUM_GLOBAL))
    loss = _run(h_src, h_dst, src_i, dst_i, etype_i, w_all, b_all, em_flat)
    return loss + jnp.asarray(inference).astype(loss.dtype) * 0.0


# speedup vs baseline: 8.9996x; 1.2473x over previous
import functools

import jax
import jax.numpy as jnp
from jax.experimental import pallas as pl
from jax.experimental.pallas import tpu as pltpu

_NUM_NODE_TYPES = 4
_NUM_PAIRS = 16
_NUM_GLOBAL = 32
_NUM_LOCAL = 8
_PAD = 128


def _fused_kernel(hs_ref, hd_ref, src_ref, dst_ref, etype_ref, w_ref, b_ref,
                  em_ref, out_ref, acc_ref, *, nblk):
    i = pl.program_id(0)

    @pl.when(i == 0)
    def _init():
        acc_ref[...] = jnp.zeros_like(acc_ref)

    h = hs_ref[...] * hd_ref[...]
    logits = jax.lax.dot_general(
        h, w_ref[...], (((1,), (0,)), ((), ())),
        preferred_element_type=jnp.float32,
        precision=jax.lax.Precision.DEFAULT) + b_ref[...]

    bsz = logits.shape[0]
    pair = src_ref[...] * _NUM_NODE_TYPES + dst_ref[...]
    col = jax.lax.broadcasted_iota(jnp.int32, (bsz, _PAD), 1)

    g = pair * _NUM_GLOBAL + etype_ref[...]
    tcol = jax.lax.broadcasted_iota(jnp.int32, (bsz, _NUM_PAIRS * _NUM_GLOBAL), 1)
    table = em_ref[0:1, :]
    label = jnp.sum(jnp.where(tcol == g, table, 0), axis=1, keepdims=True)

    base = pair * _NUM_LOCAL
    in_pair = (col >= base) & (col < base + _NUM_LOCAL)
    ml = jnp.where(in_pair, logits, -1e30)
    m = jnp.max(ml, axis=1, keepdims=True)
    lse = m + jnp.log(jnp.sum(jnp.exp(ml - m), axis=1, keepdims=True))
    picked = jnp.sum(jnp.where(col == base + label, logits, 0.0),
                     axis=1, keepdims=True)
    per_ex = lse - picked

    pbin = col == pair
    sums = jnp.sum(jnp.where(pbin, per_ex, 0.0), axis=0, keepdims=True)
    cnts = jnp.sum(pbin.astype(jnp.float32), axis=0, keepdims=True)
    acc_ref[...] += jnp.concatenate([sums, cnts], axis=0)

    @pl.when(i == nblk - 1)
    def _finish():
        tot = acc_ref[0:1, :]
        cnt = acc_ref[1:2, :]
        means = tot / jnp.maximum(cnt, 1.0)
        w = (cnt > 0.0).astype(jnp.float32)
        loss = jnp.sum(means * w) / jnp.maximum(jnp.sum(w), 1.0)
        out_ref[...] = jnp.reshape(loss, (1, 1))


@functools.partial(jax.jit, static_argnames=())
def _run(h_src, h_dst, src_i, dst_i, etype_i, w_all, b_all, em_flat):
    n, d = h_src.shape
    bsz = 2048
    nblk = n // bsz
    out = pl.pallas_call(
        functools.partial(_fused_kernel, nblk=nblk),
        grid=(nblk,),
        in_specs=[
            pl.BlockSpec((bsz, d), lambda i: (i, 0)),
            pl.BlockSpec((bsz, d), lambda i: (i, 0)),
            pl.BlockSpec((bsz, 1), lambda i: (i, 0)),
            pl.BlockSpec((bsz, 1), lambda i: (i, 0)),
            pl.BlockSpec((bsz, 1), lambda i: (i, 0)),
            pl.BlockSpec((d, _PAD), lambda i: (0, 0)),
            pl.BlockSpec((1, _PAD), lambda i: (0, 0)),
            pl.BlockSpec((8, _NUM_PAIRS * _NUM_GLOBAL), lambda i: (0, 0)),
        ],
        out_specs=pl.BlockSpec((1, 1), lambda i: (0, 0)),
        out_shape=jax.ShapeDtypeStruct((1, 1), jnp.float32),
        scratch_shapes=[pltpu.VMEM((2, _PAD), jnp.float32)],
        compiler_params=pltpu.CompilerParams(
            dimension_semantics=("arbitrary",)),
    )(h_src, h_dst, src_i, dst_i, etype_i, w_all, b_all, em_flat)
    return out[0, 0]


def kernel(h_src, h_dst, node_type_src_argmax, node_type_dst_argmax,
           edge_type_argmax, edge_type_w, edge_type_b, event_maps, inference):
    n = h_src.shape[0]
    src_i = node_type_src_argmax.astype(jnp.int32).reshape(n, 1)
    dst_i = node_type_dst_argmax.astype(jnp.int32).reshape(n, 1)
    etype_i = edge_type_argmax.astype(jnp.int32).reshape(n, 1)
    w_all = jnp.transpose(edge_type_w, (1, 0, 2)).reshape(h_src.shape[1], _PAD)
    b_all = edge_type_b.reshape(1, _PAD)
    em_flat = jnp.broadcast_to(
        event_maps.astype(jnp.int32).reshape(1, _NUM_PAIRS * _NUM_GLOBAL),
        (8, _NUM_PAIRS * _NUM_GLOBAL))
    loss = _run(h_src, h_dst, src_i, dst_i, etype_i, w_all, b_all, em_flat)
    return loss + jnp.asarray(inference).astype(loss.dtype) * 0.0
